# baseline (speedup 1.0000x reference)
import jax
import jax.numpy as jnp
from jax.experimental import pallas as pl

N, D, H, C = 10000, 128, 128, 2
BLOCK = 2000


def _mlp_mask_kernel(x_ref, w1_ref, b1_ref, alpha_ref, w2_ref, b2_ref,
                     wc_ref, bc_ref, u_ref, o_ref):
    hi = jax.lax.Precision.HIGHEST
    wc = wc_ref[...]
    wfold = jnp.dot(w2_ref[...], wc, preferred_element_type=jnp.float32,
                    precision=hi)
    bfold = jnp.dot(b2_ref[...], wc, preferred_element_type=jnp.float32,
                    precision=hi)
    h = jnp.dot(x_ref[...], w1_ref[...], preferred_element_type=jnp.float32,
                precision=hi)
    h = h + b1_ref[...]
    alpha = alpha_ref[0, 0]
    h = jnp.where(h >= 0, h, alpha * h)
    logits = jnp.dot(h, wfold, preferred_element_type=jnp.float32,
                     precision=hi)
    g = -jnp.log(-jnp.log(u_ref[...]))
    a = logits + (bfold + bc_ref[...] + g)
    win0 = (a[:, 0:1] >= a[:, 1:2]).astype(jnp.float32)
    o_ref[...] = jnp.concatenate([win0, 1.0 - win0], axis=1)


def kernel(x, edge_index, W1, b1, prelu_a, W2, b2, Wc, bc, gumbel_u):
    del edge_index
    grid = (N // BLOCK,)
    return pl.pallas_call(
        _mlp_mask_kernel,
        grid=grid,
        in_specs=[
            pl.BlockSpec((BLOCK, D), lambda i: (i, 0)),
            pl.BlockSpec((D, H), lambda i: (0, 0)),
            pl.BlockSpec((1, H), lambda i: (0, 0)),
            pl.BlockSpec((1, 1), lambda i: (0, 0)),
            pl.BlockSpec((H, H), lambda i: (0, 0)),
            pl.BlockSpec((1, H), lambda i: (0, 0)),
            pl.BlockSpec((H, C), lambda i: (0, 0)),
            pl.BlockSpec((1, C), lambda i: (0, 0)),
            pl.BlockSpec((BLOCK, C), lambda i: (i, 0)),
        ],
        out_specs=pl.BlockSpec((BLOCK, C), lambda i: (i, 0)),
        out_shape=jax.ShapeDtypeStruct((N, C), jnp.float32),
    )(x, W1, b1.reshape(1, H), prelu_a.reshape(1, 1), W2, b2.reshape(1, H),
      Wc, bc.reshape(1, C), gumbel_u)


# speedup vs baseline: 1.5285x; 1.5285x over previous
import jax
import jax.numpy as jnp
from jax.experimental import pallas as pl

N, D, H, C = 10000, 128, 128, 2
BLOCK = 2000


def _mlp_mask_kernel(x_ref, w1_ref, b1_ref, alpha_ref, w2_ref, b2_ref,
                     wc_ref, bc_ref, u_ref, o_ref):
    wc = wc_ref[...]
    wfold = jnp.dot(w2_ref[...], wc, preferred_element_type=jnp.float32)
    bfold = jnp.dot(b2_ref[...], wc, preferred_element_type=jnp.float32)
    h = jnp.dot(x_ref[...], w1_ref[...], preferred_element_type=jnp.float32)
    h = h + b1_ref[...]
    alpha = alpha_ref[0, 0]
    h = jnp.where(h >= 0, h, alpha * h)
    logits = jnp.dot(h, wfold, preferred_element_type=jnp.float32)
    g = -jnp.log(-jnp.log(u_ref[...]))
    a = logits + (bfold + bc_ref[...] + g)
    win0 = (a[:, 0:1] >= a[:, 1:2]).astype(jnp.float32)
    o_ref[...] = jnp.concatenate([win0, 1.0 - win0], axis=1)


def kernel(x, edge_index, W1, b1, prelu_a, W2, b2, Wc, bc, gumbel_u):
    del edge_index
    grid = (N // BLOCK,)
    return pl.pallas_call(
        _mlp_mask_kernel,
        grid=grid,
        in_specs=[
            pl.BlockSpec((BLOCK, D), lambda i: (i, 0)),
            pl.BlockSpec((D, H), lambda i: (0, 0)),
            pl.BlockSpec((1, H), lambda i: (0, 0)),
            pl.BlockSpec((1, 1), lambda i: (0, 0)),
            pl.BlockSpec((H, H), lambda i: (0, 0)),
            pl.BlockSpec((1, H), lambda i: (0, 0)),
            pl.BlockSpec((H, C), lambda i: (0, 0)),
            pl.BlockSpec((1, C), lambda i: (0, 0)),
            pl.BlockSpec((BLOCK, C), lambda i: (i, 0)),
        ],
        out_specs=pl.BlockSpec((BLOCK, C), lambda i: (i, 0)),
        out_shape=jax.ShapeDtypeStruct((N, C), jnp.float32),
    )(x, W1, b1.reshape(1, H), prelu_a.reshape(1, 1), W2, b2.reshape(1, H),
      Wc, bc.reshape(1, C), gumbel_u)
